# Initial kernel scaffold; baseline (speedup 1.0000x reference)
import jax
import jax.numpy as jnp
from jax.experimental import pallas as pl


def kernel(x, table, pe):
    raise NotImplementedError("write your pallas kernel here")


# speedup vs baseline: 3.0451x; 3.0451x over previous
import functools
import math

import jax
import jax.numpy as jnp
from jax import lax
from jax.experimental import pallas as pl
from jax.experimental.pallas import tpu as pltpu
from jax.experimental.pallas import tpu_sc as plsc

D = 64
NC, NS, L = 2, 16, 16
NW = NC * NS


def kernel(x, table, pe):
    B, S = x.shape
    N = B * S
    n_per_w = N // NW
    C = S
    n_chunks = n_per_w // C
    gpc = C // 128

    x1 = x.reshape(N).astype(jnp.int32)
    pe2 = pe.reshape(S, D).astype(jnp.float32)
    scale = jnp.float32(math.sqrt(D))

    mesh = plsc.VectorSubcoreMesh(
        core_axis_name="c", subcore_axis_name="s",
        num_cores=NC, num_subcores=NS)

    @functools.partial(
        pl.kernel,
        out_type=jax.ShapeDtypeStruct((N, D), jnp.float32),
        mesh=mesh,
        compiler_params=pltpu.CompilerParams(use_tc_tiling_on_sc=False),
        scratch_types=[
            pltpu.VMEM((S, D), jnp.float32),
            pltpu.VMEM((n_per_w,), jnp.int32),
            pltpu.VMEM((C, D), jnp.float32),
            pltpu.SemaphoreType.DMA,
        ],
    )
    def body(x_hbm, table_hbm, pe_hbm, out_hbm, pe_v, idx_v, rows_v, sem):
        wid = lax.axis_index("s") * NC + lax.axis_index("c")
        pltpu.sync_copy(pe_hbm, pe_v)
        pltpu.sync_copy(x_hbm.at[pl.ds(wid * n_per_w, n_per_w)], idx_v)

        @pl.loop(0, n_chunks)
        def chunk_loop(g):
            base = wid * n_per_w + g * C
            for j in range(gpc):
                pltpu.async_copy(
                    table_hbm.at[idx_v.at[pl.ds(g * C + j * 128, 128)]],
                    rows_v.at[pl.ds(j * 128, 128)], sem).wait()

            @pl.loop(0, C)
            def row_loop(r):
                for k in range(0, D, L):
                    sl = pl.ds(k, L)
                    rows_v[r, sl] = rows_v[r, sl] * scale + pe_v[r, sl]

            pltpu.sync_copy(rows_v, out_hbm.at[pl.ds(base, C)])

    out = body(x1, table, pe2)
    return out.reshape(B, S, D)


# speedup vs baseline: 3.9161x; 1.2860x over previous
import functools
import math

import jax
import jax.numpy as jnp
from jax import lax
from jax.experimental import pallas as pl
from jax.experimental.pallas import tpu as pltpu
from jax.experimental.pallas import tpu_sc as plsc

D = 64
NC, NS, L = 2, 16, 16
NW = NC * NS


def kernel(x, table, pe):
    B, S = x.shape
    N = B * S
    n_per_w = N // NW
    C = S
    n_chunks = n_per_w // C
    gpc = C // 128

    x1 = x.reshape(N).astype(jnp.int32)
    pe2 = pe.reshape(S, D).astype(jnp.float32)
    scale = jnp.float32(math.sqrt(D))

    mesh = plsc.VectorSubcoreMesh(
        core_axis_name="c", subcore_axis_name="s",
        num_cores=NC, num_subcores=NS)

    @functools.partial(
        pl.kernel,
        out_type=jax.ShapeDtypeStruct((N, D), jnp.float32),
        mesh=mesh,
        compiler_params=pltpu.CompilerParams(use_tc_tiling_on_sc=False),
        scratch_types=[
            pltpu.VMEM((S, D), jnp.float32),
            pltpu.VMEM((n_per_w,), jnp.int32),
            pltpu.VMEM((2, C, D), jnp.float32),
            pltpu.SemaphoreType.DMA((2,)),
            pltpu.SemaphoreType.DMA((2,)),
        ],
    )
    def body(x_hbm, table_hbm, pe_hbm, out_hbm, pe_v, idx_v, rows_v,
             gsem, ssem):
        wid = lax.axis_index("s") * NC + lax.axis_index("c")
        w_base = wid * n_per_w
        pltpu.sync_copy(pe_hbm, pe_v)
        pltpu.sync_copy(x_hbm.at[pl.ds(w_base, n_per_w)], idx_v)

        def fire_gather(g, b):
            for j in range(gpc):
                pltpu.async_copy(
                    table_hbm.at[idx_v.at[pl.ds(g * C + j * 128, 128)]],
                    rows_v.at[b, pl.ds(j * 128, 128)], gsem.at[b])

        fire_gather(0, 0)

        @pl.loop(0, n_chunks)
        def chunk_loop(g):
            p = lax.rem(g, 2)
            q = 1 - p

            @pl.when(g + 1 < n_chunks)
            def _fire_next():
                @pl.when(g >= 1)
                def _wait_prev_store():
                    pltpu.make_async_copy(
                        rows_v.at[q], out_hbm.at[pl.ds(0, C)],
                        ssem.at[q]).wait()
                fire_gather(g + 1, q)

            pltpu.make_async_copy(
                table_hbm.at[pl.ds(0, C)], rows_v.at[p], gsem.at[p]).wait()

            @plsc.parallel_loop(0, C, unroll=4)
            def row_loop(r):
                for k in range(0, D, L):
                    sl = pl.ds(k, L)
                    rows_v[p, r, sl] = rows_v[p, r, sl] * scale + pe_v[r, sl]

            pltpu.async_copy(rows_v.at[p],
                             out_hbm.at[pl.ds(w_base + g * C, C)], ssem.at[p])

        for b in range(2):
            pltpu.make_async_copy(
                rows_v.at[b], out_hbm.at[pl.ds(0, C)], ssem.at[b]).wait()

    out = body(x1, table, pe2)
    return out.reshape(B, S, D)
